# baseline (speedup 1.0000x reference)
import functools

import jax
import jax.numpy as jnp
from jax.experimental import pallas as pl
from jax.experimental.pallas import tpu as pltpu

N = 4096
D_IN = 512
H1 = 256
H2 = 128
NZ = 16
NC = 10
ALPHA = 0.2
V = 1.0
A = 0.5

BI = 256
GRID = N // BI
NEG = -9e15

_f32 = jnp.float32


def _relu(v):
    return jnp.maximum(v, 0.0)


def _leaky(v):
    return jnp.where(v > 0, v, ALPHA * v)


def _elu(v):
    return jnp.where(v > 0, v, jnp.exp(jnp.minimum(v, 0.0)) - 1.0)


def _dot(a, b):
    return jnp.dot(a, b, preferred_element_type=jnp.float32)


def _dot_t(a, b):
    return jax.lax.dot_general(a, b, (((1,), (1,)), ((), ())),
                               preferred_element_type=jnp.float32)


def _attention(adj_blk, h_ref, a_s, a_n, i):
    h_full = h_ref[...]
    s_row = _dot(h_ref[pl.ds(i * BI, BI), :], a_s)
    t_all = _dot(h_full, a_n)
    e = _leaky(s_row + t_all.T)
    att = jnp.where(adj_blk > 0, e, NEG)
    m = jnp.max(att, axis=1, keepdims=True)
    p = jnp.exp(att - m)
    l = jnp.sum(p, axis=1, keepdims=True)
    y = _dot(p.astype(jnp.bfloat16), h_full.astype(jnp.bfloat16)) / l
    return _elu(y)


def _struct_loss_partial(g_row, g_full, adj_blk):
    sig = jax.nn.sigmoid(_dot_t(g_row.astype(jnp.bfloat16),
                                g_full.astype(jnp.bfloat16)))
    d = sig - adj_blk
    return jnp.sum(d * d)


def _accum_loss(loss_ref, partial, i):
    p11 = jnp.reshape(partial, (1, 1))

    @pl.when(i == 0)
    def _():
        loss_ref[...] = p11

    @pl.when(i > 0)
    def _():
        loss_ref[...] += p11


def _k0(x_ref, we1, be1, we2, be2, wz, bz, wd1, bd1, wd2, bd2, wxb, bxb, wg1,
        xbar_ref, zae_ref, eh1_ref, eh2_ref, h1_ref):
    x = x_ref[...]
    eh1 = _relu(_dot(x, we1[...]) + be1[...])
    eh2 = _relu(_dot(eh1, we2[...]) + be2[...])
    zae = _dot(eh2, wz[...]) + bz[...]
    dh1 = _relu(_dot(zae, wd1[...]) + bd1[...])
    dh2 = _relu(_dot(dh1, wd2[...]) + bd2[...])
    xbar_ref[...] = _dot(dh2, wxb[...]) + bxb[...]
    zae_ref[...] = zae
    eh1_ref[...] = eh1
    eh2_ref[...] = eh2
    h1_ref[...] = _dot(x, wg1[...])


def _s1(adj_ref, h1_ref, as1, an1, eh1_ref, wg2, g1_ref, h2_ref, adjb_ref):
    i = pl.program_id(0)
    adj_blk = adj_ref[...]
    g1 = _attention(adj_blk, h1_ref, as1[...], an1[...], i)
    g1_ref[...] = g1
    xin2 = (1.0 - A) * g1 + A * eh1_ref[...]
    h2_ref[...] = _dot(xin2, wg2[...])
    adjb_ref[...] = adj_blk.astype(jnp.bfloat16)


def _s2(adj_ref, g1row_ref, g1_ref, h2_ref, as2, an2, eh2_ref, wg3,
        g2_ref, h3_ref, loss_ref):
    i = pl.program_id(0)
    adj_blk = adj_ref[...].astype(jnp.float32)
    g2 = _attention(adj_blk, h2_ref, as2[...], an2[...], i)
    g2_ref[...] = g2
    xin3 = (1.0 - A) * g2 + A * eh2_ref[...]
    h3_ref[...] = _dot(xin3, wg3[...])
    partial = _struct_loss_partial(g1row_ref[...], g1_ref[...], adj_blk)
    _accum_loss(loss_ref, partial, i)


def _s3(adj_ref, g2row_ref, g2_ref, h3_ref, as3, an3, zae_ref,
        zg_ref, zi_ref, loss_ref):
    i = pl.program_id(0)
    adj_blk = adj_ref[...].astype(jnp.float32)
    zg = _attention(adj_blk, h3_ref, as3[...], an3[...], i)
    zg_ref[...] = zg
    zi_ref[...] = (1.0 - A) * zg + A * zae_ref[...]
    partial = _struct_loss_partial(g2row_ref[...], g2_ref[...], adj_blk)
    _accum_loss(loss_ref, partial, i)


def _s4(adj_ref, zgrow_ref, zg_ref, zi_ref, ahat_ref, zl_ref, loss_ref):
    i = pl.program_id(0)
    adj_blk = adj_ref[...].astype(jnp.float32)
    ah = jax.nn.sigmoid(_dot_t(zgrow_ref[...], zg_ref[...]))
    ahat_ref[...] = ah
    d = ah - adj_blk
    _accum_loss(loss_ref, jnp.sum(d * d), i)
    zl_ref[...] = _dot(adj_blk, zi_ref[...])


def _soft_assign(z, cluster):
    zn = jnp.sum(z * z, axis=1, keepdims=True)
    cn = jnp.sum(cluster * cluster, axis=1, keepdims=True)
    d2 = zn - 2.0 * _dot_t(z, cluster) + cn.T
    q = 1.0 / (1.0 + d2 / V)
    return q / jnp.sum(q, axis=1, keepdims=True)


def _kq(zl_ref, zae_ref, cl_ref, l0_ref, l1_ref, l2_ref,
        q_ref, q1_ref, tot_ref):
    cl = cl_ref[...]
    q_ref[...] = _soft_assign(zl_ref[...], cl)
    q1_ref[...] = _soft_assign(zae_ref[...], cl)
    scale = 1.0 / (N * N)
    tot_ref[...] = (l0_ref[...] + l1_ref[...] + l2_ref[...]) * scale


def _full(shape):
    return pl.BlockSpec(shape, lambda i: (0, 0))


def _rows(width):
    return pl.BlockSpec((BI, width), lambda i: (i, 0))


def _scalar_spec():
    return pl.BlockSpec((1, 1), lambda i: (0, 0))


_SEQ = pltpu.CompilerParams(dimension_semantics=("arbitrary",))


def kernel(x, adj, W_e1, b_e1, W_e2, b_e2, W_z, b_z, W_d1, b_d1, W_d2, b_d2,
           W_xb, b_xb, Wg1, as1, an1, Wg2, as2, an2, Wg3, as3, an3,
           cluster_layer):
    f = _f32
    b2 = lambda b: b.reshape(1, -1)

    xbar, zae, eh1, eh2, h1 = pl.pallas_call(
        _k0,
        out_shape=[
            jax.ShapeDtypeStruct((N, D_IN), f),
            jax.ShapeDtypeStruct((N, NZ), f),
            jax.ShapeDtypeStruct((N, H1), f),
            jax.ShapeDtypeStruct((N, H2), f),
            jax.ShapeDtypeStruct((N, H1), f),
        ],
    )(x, W_e1, b2(b_e1), W_e2, b2(b_e2), W_z, b2(b_z), W_d1, b2(b_d1),
      W_d2, b2(b_d2), W_xb, b2(b_xb), Wg1)

    g1, h2, adjb = pl.pallas_call(
        _s1,
        grid=(GRID,),
        in_specs=[_rows(N), _full((N, H1)), _full((H1, 1)), _full((H1, 1)),
                  _rows(H1), _full((H1, H2))],
        out_specs=[_rows(H1), _rows(H2), _rows(N)],
        out_shape=[jax.ShapeDtypeStruct((N, H1), f),
                   jax.ShapeDtypeStruct((N, H2), f),
                   jax.ShapeDtypeStruct((N, N), jnp.bfloat16)],
        compiler_params=_SEQ,
    )(adj, h1, as1, an1, eh1, Wg2)

    g2, h3, l0 = pl.pallas_call(
        _s2,
        grid=(GRID,),
        in_specs=[_rows(N), _rows(H1), _full((N, H1)), _full((N, H2)),
                  _full((H2, 1)), _full((H2, 1)), _rows(H2), _full((H2, NZ))],
        out_specs=[_rows(H2), _rows(NZ), _scalar_spec()],
        out_shape=[jax.ShapeDtypeStruct((N, H2), f),
                   jax.ShapeDtypeStruct((N, NZ), f),
                   jax.ShapeDtypeStruct((1, 1), f)],
        compiler_params=_SEQ,
    )(adjb, g1, g1, h2, as2, an2, eh2, Wg3)

    zg, zi, l1 = pl.pallas_call(
        _s3,
        grid=(GRID,),
        in_specs=[_rows(N), _rows(H2), _full((N, H2)), _full((N, NZ)),
                  _full((NZ, 1)), _full((NZ, 1)), _rows(NZ)],
        out_specs=[_rows(NZ), _rows(NZ), _scalar_spec()],
        out_shape=[jax.ShapeDtypeStruct((N, NZ), f),
                   jax.ShapeDtypeStruct((N, NZ), f),
                   jax.ShapeDtypeStruct((1, 1), f)],
        compiler_params=_SEQ,
    )(adjb, g2, g2, h3, as3, an3, zae)

    ahat, zl, l2 = pl.pallas_call(
        _s4,
        grid=(GRID,),
        in_specs=[_rows(N), _rows(NZ), _full((N, NZ)), _full((N, NZ))],
        out_specs=[_rows(N), _rows(NZ), _scalar_spec()],
        out_shape=[jax.ShapeDtypeStruct((N, N), f),
                   jax.ShapeDtypeStruct((N, NZ), f),
                   jax.ShapeDtypeStruct((1, 1), f)],
        compiler_params=_SEQ,
    )(adjb, zg, zg, zi)

    q, q1, tot = pl.pallas_call(
        _kq,
        out_shape=[jax.ShapeDtypeStruct((N, NC), f),
                   jax.ShapeDtypeStruct((N, NC), f),
                   jax.ShapeDtypeStruct((1, 1), f)],
    )(zl, zae, cluster_layer, l0, l1, l2)

    return (xbar, ahat, zae, q, q1, zl, tot.reshape(()))


# speedup vs baseline: 1.0174x; 1.0174x over previous
import jax
import jax.numpy as jnp
from jax.experimental import pallas as pl
from jax.experimental.pallas import tpu as pltpu

N = 4096
D_IN = 512
H1 = 256
H2 = 128
NZ = 16
NC = 10
ALPHA = 0.2
V = 1.0
A = 0.5

BI = 256
GRID = N // BI
NEG = -9e15

_f32 = jnp.float32
_bf16 = jnp.bfloat16


def _relu(v):
    return jnp.maximum(v, 0.0)


def _leaky(v):
    return jnp.maximum(v, ALPHA * v)


def _elu(v):
    return jnp.where(v > 0, v, jnp.exp(jnp.minimum(v, 0.0)) - 1.0)


def _dot(a, b):
    return jnp.dot(a, b, preferred_element_type=jnp.float32)


def _dot_t(a, b):
    return jax.lax.dot_general(a, b, (((1,), (1,)), ((), ())),
                               preferred_element_type=jnp.float32)


def _attention(adj_blk, hf_ref, a_s, a_n, i):
    h_full = hf_ref[...]
    s_row = _dot(hf_ref[pl.ds(i * BI, BI), :], a_s)
    t_all = _dot(h_full, a_n)
    m = _leaky(s_row + jnp.max(t_all))
    e = _leaky(s_row + t_all.T)
    att = jnp.where(adj_blk > 0, e, NEG)
    p = jnp.exp(att - m)
    l = jnp.sum(p, axis=1, keepdims=True)
    l = jnp.maximum(l, 1e-30)
    y = _dot(p, h_full) / l
    return _elu(y)


def _struct_loss_partial(neg_g_row, g_full, adj_blk):
    u = jnp.exp(_dot_t(neg_g_row, g_full))
    d = 1.0 / (1.0 + u) - adj_blk
    return jnp.sum(d * d)


def _accum_loss(loss_ref, partial, i):
    p11 = jnp.reshape(partial, (1, 1))

    @pl.when(i == 0)
    def _():
        loss_ref[...] = p11

    @pl.when(i > 0)
    def _():
        loss_ref[...] += p11


def _k0(x_ref, we1, be1, we2, be2, wz, bz, wd1, bd1, wd2, bd2, wxb, bxb, wg1,
        xbar_ref, zae_ref, eh1_ref, eh2_ref, h1_ref):
    x = x_ref[...]
    eh1 = _relu(_dot(x, we1[...]) + be1[...])
    eh2 = _relu(_dot(eh1, we2[...]) + be2[...])
    zae = _dot(eh2, wz[...]) + bz[...]
    dh1 = _relu(_dot(zae, wd1[...]) + bd1[...])
    dh2 = _relu(_dot(dh1, wd2[...]) + bd2[...])
    xbar_ref[...] = _dot(dh2, wxb[...]) + bxb[...]
    zae_ref[...] = zae
    eh1_ref[...] = eh1
    eh2_ref[...] = eh2
    h1_ref[...] = _dot(x, wg1[...])


def _s1(adj_ref, h1_ref, as1, an1, eh1_ref, wg2, g1b_ref, h2_ref):
    i = pl.program_id(0)
    g1 = _attention(adj_ref[...], h1_ref, as1[...], an1[...], i)
    g1b_ref[...] = g1.astype(_bf16)
    xin2 = (1.0 - A) * g1 + A * eh1_ref[...]
    h2_ref[...] = _dot(xin2, wg2[...])


def _s2(adj_ref, g1b_row_ref, g1b_ref, h2_ref, as2, an2, eh2_ref, wg3,
        g2b_ref, h3_ref, loss_ref):
    i = pl.program_id(0)
    adj_blk = adj_ref[...]
    g2 = _attention(adj_blk, h2_ref, as2[...], an2[...], i)
    g2b_ref[...] = g2.astype(_bf16)
    xin3 = (1.0 - A) * g2 + A * eh2_ref[...]
    h3_ref[...] = _dot(xin3, wg3[...])
    partial = _struct_loss_partial(-g1b_row_ref[...], g1b_ref[...], adj_blk)
    _accum_loss(loss_ref, partial, i)


def _s3(adj_ref, g2b_row_ref, g2b_ref, h3_ref, as3, an3, zae_ref,
        zg_ref, zi_ref, loss_ref):
    i = pl.program_id(0)
    adj_blk = adj_ref[...]
    zg = _attention(adj_blk, h3_ref, as3[...], an3[...], i)
    zg_ref[...] = zg
    zi_ref[...] = (1.0 - A) * zg + A * zae_ref[...]
    partial = _struct_loss_partial(-g2b_row_ref[...], g2b_ref[...], adj_blk)
    _accum_loss(loss_ref, partial, i)


def _s4(adj_ref, zgrow_ref, zg_ref, zi_ref, ahat_ref, zl_ref, loss_ref):
    i = pl.program_id(0)
    adj_blk = adj_ref[...]
    u = jnp.exp(_dot_t(-zgrow_ref[...], zg_ref[...]))
    ah = 1.0 / (1.0 + u)
    ahat_ref[...] = ah
    d = ah - adj_blk
    _accum_loss(loss_ref, jnp.sum(d * d), i)
    zl_ref[...] = _dot(adj_blk, zi_ref[...])


def _soft_assign(z, cluster):
    zn = jnp.sum(z * z, axis=1, keepdims=True)
    cn = jnp.sum(cluster * cluster, axis=1, keepdims=True)
    d2 = zn - 2.0 * _dot_t(z, cluster) + cn.T
    q = 1.0 / (1.0 + d2 / V)
    return q / jnp.sum(q, axis=1, keepdims=True)


def _kq(zl_ref, zae_ref, cl_ref, l0_ref, l1_ref, l2_ref,
        q_ref, q1_ref, tot_ref):
    cl = cl_ref[...]
    q_ref[...] = _soft_assign(zl_ref[...], cl)
    q1_ref[...] = _soft_assign(zae_ref[...], cl)
    scale = 1.0 / (N * N)
    tot_ref[...] = (l0_ref[...] + l1_ref[...] + l2_ref[...]) * scale


def _full(shape):
    return pl.BlockSpec(shape, lambda i: (0, 0))


def _rows(width):
    return pl.BlockSpec((BI, width), lambda i: (i, 0))


def _scalar_spec():
    return pl.BlockSpec((1, 1), lambda i: (0, 0))


_SEQ = pltpu.CompilerParams(dimension_semantics=("arbitrary",))


def kernel(x, adj, W_e1, b_e1, W_e2, b_e2, W_z, b_z, W_d1, b_d1, W_d2, b_d2,
           W_xb, b_xb, Wg1, as1, an1, Wg2, as2, an2, Wg3, as3, an3,
           cluster_layer):
    f = _f32
    b2 = lambda b: b.reshape(1, -1)

    xbar, zae, eh1, eh2, h1 = pl.pallas_call(
        _k0,
        out_shape=[
            jax.ShapeDtypeStruct((N, D_IN), f),
            jax.ShapeDtypeStruct((N, NZ), f),
            jax.ShapeDtypeStruct((N, H1), f),
            jax.ShapeDtypeStruct((N, H2), f),
            jax.ShapeDtypeStruct((N, H1), f),
        ],
    )(x, W_e1, b2(b_e1), W_e2, b2(b_e2), W_z, b2(b_z), W_d1, b2(b_d1),
      W_d2, b2(b_d2), W_xb, b2(b_xb), Wg1)

    g1b, h2 = pl.pallas_call(
        _s1,
        grid=(GRID,),
        in_specs=[_rows(N), _full((N, H1)), _full((H1, 1)), _full((H1, 1)),
                  _rows(H1), _full((H1, H2))],
        out_specs=[_rows(H1), _rows(H2)],
        out_shape=[jax.ShapeDtypeStruct((N, H1), _bf16),
                   jax.ShapeDtypeStruct((N, H2), f)],
        compiler_params=_SEQ,
    )(adj, h1, as1, an1, eh1, Wg2)

    g2b, h3, l0 = pl.pallas_call(
        _s2,
        grid=(GRID,),
        in_specs=[_rows(N), _rows(H1), _full((N, H1)), _full((N, H2)),
                  _full((H2, 1)), _full((H2, 1)), _rows(H2), _full((H2, NZ))],
        out_specs=[_rows(H2), _rows(NZ), _scalar_spec()],
        out_shape=[jax.ShapeDtypeStruct((N, H2), _bf16),
                   jax.ShapeDtypeStruct((N, NZ), f),
                   jax.ShapeDtypeStruct((1, 1), f)],
        compiler_params=_SEQ,
    )(adj, g1b, g1b, h2, as2, an2, eh2, Wg3)

    zg, zi, l1 = pl.pallas_call(
        _s3,
        grid=(GRID,),
        in_specs=[_rows(N), _rows(H2), _full((N, H2)), _full((N, NZ)),
                  _full((NZ, 1)), _full((NZ, 1)), _rows(NZ)],
        out_specs=[_rows(NZ), _rows(NZ), _scalar_spec()],
        out_shape=[jax.ShapeDtypeStruct((N, NZ), f),
                   jax.ShapeDtypeStruct((N, NZ), f),
                   jax.ShapeDtypeStruct((1, 1), f)],
        compiler_params=_SEQ,
    )(adj, g2b, g2b, h3, as3, an3, zae)

    ahat, zl, l2 = pl.pallas_call(
        _s4,
        grid=(GRID,),
        in_specs=[_rows(N), _rows(NZ), _full((N, NZ)), _full((N, NZ))],
        out_specs=[_rows(N), _rows(NZ), _scalar_spec()],
        out_shape=[jax.ShapeDtypeStruct((N, N), f),
                   jax.ShapeDtypeStruct((N, NZ), f),
                   jax.ShapeDtypeStruct((1, 1), f)],
        compiler_params=_SEQ,
    )(adj, zg, zg, zi)

    q, q1, tot = pl.pallas_call(
        _kq,
        out_shape=[jax.ShapeDtypeStruct((N, NC), f),
                   jax.ShapeDtypeStruct((N, NC), f),
                   jax.ShapeDtypeStruct((1, 1), f)],
    )(zl, zae, cluster_layer, l0, l1, l2)

    return (xbar, ahat, zae, q, q1, zl, tot.reshape(()))


# speedup vs baseline: 1.0790x; 1.0606x over previous
import jax
import jax.numpy as jnp
from jax.experimental import pallas as pl
from jax.experimental.pallas import tpu as pltpu

N = 4096
D_IN = 512
H1 = 256
H2 = 128
NZ = 16
NC = 10
ALPHA = 0.2
V = 1.0
A = 0.5

BI = 512
GRID = N // BI
NEG = -9e15

_f32 = jnp.float32
_bf16 = jnp.bfloat16


def _relu(v):
    return jnp.maximum(v, 0.0)


def _leaky(v):
    return jnp.maximum(v, ALPHA * v)


def _elu(v):
    return jnp.where(v > 0, v, jnp.exp(jnp.minimum(v, 0.0)) - 1.0)


def _dot(a, b):
    return jnp.dot(a, b, preferred_element_type=jnp.float32)


def _dot_t(a, b):
    return jax.lax.dot_general(a, b, (((1,), (1,)), ((), ())),
                               preferred_element_type=jnp.float32)


def _attention(adj_blk, hf_ref, a_s, a_n, i):
    h_full = hf_ref[...]
    s_row = _dot(hf_ref[pl.ds(i * BI, BI), :], a_s)
    t_all = _dot(h_full, a_n)
    m = _leaky(s_row + jnp.max(t_all))
    e = _leaky(s_row + t_all.T)
    att = jnp.where(adj_blk > 0, e, NEG)
    p = jnp.exp(att - m)
    l = jnp.sum(p, axis=1, keepdims=True)
    l = jnp.maximum(l, 1e-30)
    y = _dot(p, h_full) / l
    return _elu(y)


def _struct_loss_partial(neg_g_row, g_full, adj_blk):
    u = jnp.exp(_dot_t(neg_g_row, g_full))
    d = 1.0 / (1.0 + u) - adj_blk
    return jnp.sum(d * d)


def _accum_loss(loss_ref, partial, i):
    p11 = jnp.reshape(partial, (1, 1))

    @pl.when(i == 0)
    def _():
        loss_ref[...] = p11

    @pl.when(i > 0)
    def _():
        loss_ref[...] += p11


def _k0(x_ref, we1, be1, we2, be2, wz, bz, wd1, bd1, wd2, bd2, wxb, bxb, wg1,
        xbar_ref, zae_ref, eh1_ref, eh2_ref, h1_ref):
    x = x_ref[...]
    eh1 = _relu(_dot(x, we1[...]) + be1[...])
    eh2 = _relu(_dot(eh1, we2[...]) + be2[...])
    zae = _dot(eh2, wz[...]) + bz[...]
    dh1 = _relu(_dot(zae, wd1[...]) + bd1[...])
    dh2 = _relu(_dot(dh1, wd2[...]) + bd2[...])
    xbar_ref[...] = _dot(dh2, wxb[...]) + bxb[...]
    zae_ref[...] = zae
    eh1_ref[...] = eh1
    eh2_ref[...] = eh2
    h1_ref[...] = _dot(x, wg1[...])


def _s1(adj_ref, h1_ref, as1, an1, eh1_ref, wg2, g1b_ref, h2_ref):
    i = pl.program_id(0)
    g1 = _attention(adj_ref[...], h1_ref, as1[...], an1[...], i)
    g1b_ref[...] = g1.astype(_bf16)
    xin2 = (1.0 - A) * g1 + A * eh1_ref[...]
    h2_ref[...] = _dot(xin2, wg2[...])


def _s2(adj_ref, g1b_row_ref, g1b_ref, h2_ref, as2, an2, eh2_ref, wg3,
        g2b_ref, h3_ref, loss_ref):
    i = pl.program_id(0)
    adj_blk = adj_ref[...]
    g2 = _attention(adj_blk, h2_ref, as2[...], an2[...], i)
    g2b_ref[...] = g2.astype(_bf16)
    xin3 = (1.0 - A) * g2 + A * eh2_ref[...]
    h3_ref[...] = _dot(xin3, wg3[...])
    partial = _struct_loss_partial(-g1b_row_ref[...], g1b_ref[...], adj_blk)
    _accum_loss(loss_ref, partial, i)


def _s3(adj_ref, g2b_row_ref, g2b_ref, h3_ref, as3, an3, zae_ref,
        zg_ref, zi_ref, loss_ref):
    i = pl.program_id(0)
    adj_blk = adj_ref[...]
    zg = _attention(adj_blk, h3_ref, as3[...], an3[...], i)
    zg_ref[...] = zg
    zi_ref[...] = (1.0 - A) * zg + A * zae_ref[...]
    partial = _struct_loss_partial(-g2b_row_ref[...], g2b_ref[...], adj_blk)
    _accum_loss(loss_ref, partial, i)


def _s4(adj_ref, zgrow_ref, zg_ref, zi_ref, ahat_ref, zl_ref, loss_ref):
    i = pl.program_id(0)
    adj_blk = adj_ref[...]
    u = jnp.exp(_dot_t(-zgrow_ref[...], zg_ref[...]))
    ah = 1.0 / (1.0 + u)
    ahat_ref[...] = ah
    d = ah - adj_blk
    _accum_loss(loss_ref, jnp.sum(d * d), i)
    zl_ref[...] = _dot(adj_blk, zi_ref[...])


def _soft_assign(z, cluster):
    zn = jnp.sum(z * z, axis=1, keepdims=True)
    cn = jnp.sum(cluster * cluster, axis=1, keepdims=True)
    d2 = zn - 2.0 * _dot_t(z, cluster) + cn.T
    q = 1.0 / (1.0 + d2 / V)
    return q / jnp.sum(q, axis=1, keepdims=True)


def _kq(zl_ref, zae_ref, cl_ref, l0_ref, l1_ref, l2_ref,
        q_ref, q1_ref, tot_ref):
    cl = cl_ref[...]
    q_ref[...] = _soft_assign(zl_ref[...], cl)
    q1_ref[...] = _soft_assign(zae_ref[...], cl)
    scale = 1.0 / (N * N)
    tot_ref[...] = (l0_ref[...] + l1_ref[...] + l2_ref[...]) * scale


def _full(shape):
    return pl.BlockSpec(shape, lambda i: (0, 0))


def _rows(width):
    return pl.BlockSpec((BI, width), lambda i: (i, 0))


def _scalar_spec():
    return pl.BlockSpec((1, 1), lambda i: (0, 0))


_SEQ = pltpu.CompilerParams(dimension_semantics=("arbitrary",))


def kernel(x, adj, W_e1, b_e1, W_e2, b_e2, W_z, b_z, W_d1, b_d1, W_d2, b_d2,
           W_xb, b_xb, Wg1, as1, an1, Wg2, as2, an2, Wg3, as3, an3,
           cluster_layer):
    f = _f32
    b2 = lambda b: b.reshape(1, -1)

    xbar, zae, eh1, eh2, h1 = pl.pallas_call(
        _k0,
        out_shape=[
            jax.ShapeDtypeStruct((N, D_IN), f),
            jax.ShapeDtypeStruct((N, NZ), f),
            jax.ShapeDtypeStruct((N, H1), f),
            jax.ShapeDtypeStruct((N, H2), f),
            jax.ShapeDtypeStruct((N, H1), f),
        ],
    )(x, W_e1, b2(b_e1), W_e2, b2(b_e2), W_z, b2(b_z), W_d1, b2(b_d1),
      W_d2, b2(b_d2), W_xb, b2(b_xb), Wg1)

    g1b, h2 = pl.pallas_call(
        _s1,
        grid=(GRID,),
        in_specs=[_rows(N), _full((N, H1)), _full((H1, 1)), _full((H1, 1)),
                  _rows(H1), _full((H1, H2))],
        out_specs=[_rows(H1), _rows(H2)],
        out_shape=[jax.ShapeDtypeStruct((N, H1), _bf16),
                   jax.ShapeDtypeStruct((N, H2), f)],
        compiler_params=_SEQ,
    )(adj, h1, as1, an1, eh1, Wg2)

    g2b, h3, l0 = pl.pallas_call(
        _s2,
        grid=(GRID,),
        in_specs=[_rows(N), _rows(H1), _full((N, H1)), _full((N, H2)),
                  _full((H2, 1)), _full((H2, 1)), _rows(H2), _full((H2, NZ))],
        out_specs=[_rows(H2), _rows(NZ), _scalar_spec()],
        out_shape=[jax.ShapeDtypeStruct((N, H2), _bf16),
                   jax.ShapeDtypeStruct((N, NZ), f),
                   jax.ShapeDtypeStruct((1, 1), f)],
        compiler_params=_SEQ,
    )(adj, g1b, g1b, h2, as2, an2, eh2, Wg3)

    zg, zi, l1 = pl.pallas_call(
        _s3,
        grid=(GRID,),
        in_specs=[_rows(N), _rows(H2), _full((N, H2)), _full((N, NZ)),
                  _full((NZ, 1)), _full((NZ, 1)), _rows(NZ)],
        out_specs=[_rows(NZ), _rows(NZ), _scalar_spec()],
        out_shape=[jax.ShapeDtypeStruct((N, NZ), f),
                   jax.ShapeDtypeStruct((N, NZ), f),
                   jax.ShapeDtypeStruct((1, 1), f)],
        compiler_params=_SEQ,
    )(adj, g2b, g2b, h3, as3, an3, zae)

    ahat, zl, l2 = pl.pallas_call(
        _s4,
        grid=(GRID,),
        in_specs=[_rows(N), _rows(NZ), _full((N, NZ)), _full((N, NZ))],
        out_specs=[_rows(N), _rows(NZ), _scalar_spec()],
        out_shape=[jax.ShapeDtypeStruct((N, N), f),
                   jax.ShapeDtypeStruct((N, NZ), f),
                   jax.ShapeDtypeStruct((1, 1), f)],
        compiler_params=_SEQ,
    )(adj, zg, zg, zi)

    q, q1, tot = pl.pallas_call(
        _kq,
        out_shape=[jax.ShapeDtypeStruct((N, NC), f),
                   jax.ShapeDtypeStruct((N, NC), f),
                   jax.ShapeDtypeStruct((1, 1), f)],
    )(zl, zae, cluster_layer, l0, l1, l2)

    return (xbar, ahat, zae, q, q1, zl, tot.reshape(()))
